# baseline (speedup 1.0000x reference)
import functools

import jax
import jax.numpy as jnp
from jax.experimental import pallas as pl
from jax.experimental.pallas import tpu as pltpu

_K = 32
_T = 256
_BIG = 3e38


def _stage1_kernel(xt_ref, xf_ref, wbt_ref, minv_ref, s_ref, q_ref, usc):
    xt = xt_ref[0]
    xf = xf_ref[0]
    T = xt.shape[0]
    N = xf.shape[0]
    x2t = jnp.sum(xt * xt, axis=1, keepdims=True)
    x2f = jnp.sum(xf * xf, axis=1, keepdims=True)
    cross = jax.lax.dot_general(
        xt, xf, (((1,), (1,)), ((), ())),
        preferred_element_type=jnp.float32)
    d = x2t + jnp.transpose(x2f) - 2.0 * cross
    bits = jax.lax.bitcast_convert_type(d, jnp.int32)
    u = bits ^ ((bits >> 31) & jnp.int32(0x7FFFFFFF))
    usc[...] = u
    f = u
    w = N
    while w > 2 * _K:
        w //= 2
        f = jnp.minimum(f[:, :w], f[:, w:])
    lo = jnp.min(f, axis=1, keepdims=True) - 1
    hi = jnp.max(f, axis=1, keepdims=True)
    kf = jnp.float32(_K)

    def avg(a, b):
        return (a >> 1) + (b >> 1) + (a & b & 1)

    def cond(carry):
        i, lo, hi = carry
        return jnp.logical_or(i < 2, jnp.any((hi - lo) > 1))

    def body(carry):
        i, lo, hi = carry
        m2 = avg(lo, hi)
        m1 = avg(lo, m2)
        m3 = avg(m2, hi)
        uu = usc[...]
        c1 = jnp.sum((uu <= m1).astype(jnp.float32), axis=1, keepdims=True)
        c2 = jnp.sum((uu <= m2).astype(jnp.float32), axis=1, keepdims=True)
        c3 = jnp.sum((uu <= m3).astype(jnp.float32), axis=1, keepdims=True)
        hi = jnp.where(c1 >= kf, m1, jnp.where(c2 >= kf, m2,
                       jnp.where(c3 >= kf, m3, hi)))
        lo = jnp.where(c3 < kf, m3, jnp.where(c2 < kf, m2,
                       jnp.where(c1 < kf, m1, lo)))
        return i + 1, lo, hi

    _, lo, hi = jax.lax.while_loop(cond, body, (jnp.int32(0), lo, hi))
    t = hi
    M = (usc[...] <= t).astype(jnp.float32)
    vT = jax.lax.dot_general(
        wbt_ref[...], xf, (((1,), (1,)), ((), ())),
        preferred_element_type=jnp.float32)
    s_ref[0] = jax.lax.dot_general(
        M, vT, (((1,), (1,)), ((), ())),
        preferred_element_type=jnp.float32)
    q_ref[0] = jax.lax.dot_general(
        M, vT * vT, (((1,), (1,)), ((), ())),
        preferred_element_type=jnp.float32)
    keep = M > 0.0
    cols = []
    for c in range(vT.shape[0]):
        vc = jax.lax.slice_in_dim(vT, c, c + 1, axis=0)
        cols.append(jnp.min(jnp.where(keep, vc, _BIG), axis=1, keepdims=True))
    minv_ref[0] = jnp.concatenate(cols, axis=1)


def _stage2_kernel(x_ref, wab_ref, b1_ref, g1_ref, be1_ref, w2_ref, b2_ref,
                   g2_ref, be2_ref, minv_ref, s_ref, q_ref, out_ref):
    x = x_ref[...]
    u = jax.lax.dot_general(
        x, wab_ref[...], (((1,), (1,)), ((), ())),
        preferred_element_type=jnp.float32) + b1_ref[...]
    s = s_ref[...]
    q = q_ref[...]
    minv = minv_ref[...]
    cnt = jnp.float32(x.shape[0] * _K)
    kf = jnp.float32(_K)
    sum_h = kf * jnp.sum(u, axis=0, keepdims=True) - jnp.sum(
        s, axis=0, keepdims=True)
    sumsq_h = jnp.sum(kf * u * u - 2.0 * u * s + q, axis=0, keepdims=True)
    mean = sum_h / cnt
    var = sumsq_h / cnt - mean * mean
    mpool = u - minv
    hn = (mpool - mean) / jnp.sqrt(var + 1e-5)
    hn = hn * g1_ref[...] + be1_ref[...]
    e = jnp.where(hn > 0.0, hn, jnp.exp(hn) - 1.0)
    y = jax.lax.dot_general(
        e, w2_ref[...], (((1,), (1,)), ((), ())),
        preferred_element_type=jnp.float32) + b2_ref[...]
    mean2 = jnp.mean(y, axis=0, keepdims=True)
    var2 = jnp.mean((y - mean2) ** 2, axis=0, keepdims=True)
    yn = (y - mean2) / jnp.sqrt(var2 + 1e-5)
    yn = yn * g2_ref[...] + be2_ref[...]
    out_ref[...] = jnp.where(yn > 0.0, yn, jnp.exp(yn) - 1.0)


@jax.jit
def kernel(x, W1, b1, g1, be1, W2, b2, g2, be2):
    B, N, C = x.shape
    C2 = W1.shape[0]
    OUT = W2.shape[0]
    T = _T
    wb = W1[:, C:]
    wab = W1[:, :C] + wb

    grid = (B, N // T)
    minv, s, q = pl.pallas_call(
        _stage1_kernel,
        grid=grid,
        in_specs=[
            pl.BlockSpec((1, T, C), lambda b, t: (b, t, 0)),
            pl.BlockSpec((1, N, C), lambda b, t: (b, 0, 0)),
            pl.BlockSpec((C2, C), lambda b, t: (0, 0)),
        ],
        out_specs=[
            pl.BlockSpec((1, T, C2), lambda b, t: (b, t, 0)),
            pl.BlockSpec((1, T, C2), lambda b, t: (b, t, 0)),
            pl.BlockSpec((1, T, C2), lambda b, t: (b, t, 0)),
        ],
        out_shape=[
            jax.ShapeDtypeStruct((B, N, C2), jnp.float32),
            jax.ShapeDtypeStruct((B, N, C2), jnp.float32),
            jax.ShapeDtypeStruct((B, N, C2), jnp.float32),
        ],
        scratch_shapes=[
            pltpu.VMEM((T, N), jnp.int32),
        ],
    )(x, x, wb)

    BN = B * N
    out = pl.pallas_call(
        _stage2_kernel,
        out_shape=jax.ShapeDtypeStruct((BN, OUT), jnp.float32),
    )(x.reshape(BN, C), wab, b1.reshape(1, C2), g1.reshape(1, C2),
      be1.reshape(1, C2), W2, b2.reshape(1, OUT), g2.reshape(1, OUT),
      be2.reshape(1, OUT), minv.reshape(BN, C2), s.reshape(BN, C2),
      q.reshape(BN, C2))
    return out.reshape(B, N, OUT)


# speedup vs baseline: 1.0909x; 1.0909x over previous
import functools

import jax
import jax.numpy as jnp
from jax.experimental import pallas as pl
from jax.experimental.pallas import tpu as pltpu

_K = 32
_T = 256
_BIG = 3e38


def _stage1_kernel(xt_ref, xf_ref, wbt_ref, minv_ref, s_ref, q_ref, usc):
    xt = xt_ref[0]
    xf = xf_ref[0]
    T = xt.shape[0]
    N = xf.shape[0]
    x2t = jnp.sum(xt * xt, axis=1, keepdims=True)
    x2f = jnp.sum(xf * xf, axis=1, keepdims=True)
    cross = jax.lax.dot_general(
        xt, xf, (((1,), (1,)), ((), ())),
        preferred_element_type=jnp.float32)
    d = x2t + jnp.transpose(x2f) - 2.0 * cross
    bits = jax.lax.bitcast_convert_type(d, jnp.int32)
    u = bits ^ ((bits >> 31) & jnp.int32(0x7FFFFFFF))
    usc[...] = u
    f = u
    w = N
    while w > 2 * _K:
        w //= 2
        f = jnp.minimum(f[:, :w], f[:, w:])
    lo = jnp.min(f, axis=1, keepdims=True) - 1
    hi = jnp.max(f, axis=1, keepdims=True)
    kf = jnp.float32(_K)

    def avg(a, b):
        return (a >> 1) + (b >> 1) + (a & b & 1)

    def body(carry):
        i, lo, hi = carry
        m2 = avg(lo, hi)
        m1 = avg(lo, m2)
        m3 = avg(m2, hi)
        uu = usc[...]
        c1 = jnp.sum((uu <= m1).astype(jnp.float32), axis=1, keepdims=True)
        c2 = jnp.sum((uu <= m2).astype(jnp.float32), axis=1, keepdims=True)
        c3 = jnp.sum((uu <= m3).astype(jnp.float32), axis=1, keepdims=True)
        hi = jnp.where(c1 >= kf, m1, jnp.where(c2 >= kf, m2,
                       jnp.where(c3 >= kf, m3, hi)))
        lo = jnp.where(c3 < kf, m3, jnp.where(c2 < kf, m2,
                       jnp.where(c1 < kf, m1, lo)))
        return i + 1, lo, hi

    carry = (jnp.int32(0), lo, hi)
    for _ in range(12):
        carry = body(carry)

    def cond(carry):
        i, lo, hi = carry
        return jnp.any((hi - lo) > 1)

    _, lo, hi = jax.lax.while_loop(cond, body, carry)
    t = hi
    M = (usc[...] <= t).astype(jnp.float32)
    vT = jax.lax.dot_general(
        wbt_ref[...], xf, (((1,), (1,)), ((), ())),
        preferred_element_type=jnp.float32)
    s_ref[0] = jax.lax.dot_general(
        M, vT, (((1,), (1,)), ((), ())),
        preferred_element_type=jnp.float32)
    q_ref[0] = jax.lax.dot_general(
        M, vT * vT, (((1,), (1,)), ((), ())),
        preferred_element_type=jnp.float32)
    keep = M > 0.0
    cols = []
    for c in range(vT.shape[0]):
        vc = jax.lax.slice_in_dim(vT, c, c + 1, axis=0)
        cols.append(jnp.min(jnp.where(keep, vc, _BIG), axis=1, keepdims=True))
    minv_ref[0] = jnp.concatenate(cols, axis=1)


def _stage2_kernel(x_ref, wab_ref, b1_ref, g1_ref, be1_ref, w2_ref, b2_ref,
                   g2_ref, be2_ref, minv_ref, s_ref, q_ref, out_ref):
    x = x_ref[...]
    u = jax.lax.dot_general(
        x, wab_ref[...], (((1,), (1,)), ((), ())),
        preferred_element_type=jnp.float32) + b1_ref[...]
    s = s_ref[...]
    q = q_ref[...]
    minv = minv_ref[...]
    cnt = jnp.float32(x.shape[0] * _K)
    kf = jnp.float32(_K)
    sum_h = kf * jnp.sum(u, axis=0, keepdims=True) - jnp.sum(
        s, axis=0, keepdims=True)
    sumsq_h = jnp.sum(kf * u * u - 2.0 * u * s + q, axis=0, keepdims=True)
    mean = sum_h / cnt
    var = sumsq_h / cnt - mean * mean
    mpool = u - minv
    hn = (mpool - mean) / jnp.sqrt(var + 1e-5)
    hn = hn * g1_ref[...] + be1_ref[...]
    e = jnp.where(hn > 0.0, hn, jnp.exp(hn) - 1.0)
    y = jax.lax.dot_general(
        e, w2_ref[...], (((1,), (1,)), ((), ())),
        preferred_element_type=jnp.float32) + b2_ref[...]
    mean2 = jnp.mean(y, axis=0, keepdims=True)
    var2 = jnp.mean((y - mean2) ** 2, axis=0, keepdims=True)
    yn = (y - mean2) / jnp.sqrt(var2 + 1e-5)
    yn = yn * g2_ref[...] + be2_ref[...]
    out_ref[...] = jnp.where(yn > 0.0, yn, jnp.exp(yn) - 1.0)


@jax.jit
def kernel(x, W1, b1, g1, be1, W2, b2, g2, be2):
    B, N, C = x.shape
    C2 = W1.shape[0]
    OUT = W2.shape[0]
    T = _T
    wb = W1[:, C:]
    wab = W1[:, :C] + wb

    grid = (B, N // T)
    minv, s, q = pl.pallas_call(
        _stage1_kernel,
        grid=grid,
        in_specs=[
            pl.BlockSpec((1, T, C), lambda b, t: (b, t, 0)),
            pl.BlockSpec((1, N, C), lambda b, t: (b, 0, 0)),
            pl.BlockSpec((C2, C), lambda b, t: (0, 0)),
        ],
        out_specs=[
            pl.BlockSpec((1, T, C2), lambda b, t: (b, t, 0)),
            pl.BlockSpec((1, T, C2), lambda b, t: (b, t, 0)),
            pl.BlockSpec((1, T, C2), lambda b, t: (b, t, 0)),
        ],
        out_shape=[
            jax.ShapeDtypeStruct((B, N, C2), jnp.float32),
            jax.ShapeDtypeStruct((B, N, C2), jnp.float32),
            jax.ShapeDtypeStruct((B, N, C2), jnp.float32),
        ],
        scratch_shapes=[
            pltpu.VMEM((T, N), jnp.int32),
        ],
    )(x, x, wb)

    BN = B * N
    out = pl.pallas_call(
        _stage2_kernel,
        out_shape=jax.ShapeDtypeStruct((BN, OUT), jnp.float32),
    )(x.reshape(BN, C), wab, b1.reshape(1, C2), g1.reshape(1, C2),
      be1.reshape(1, C2), W2, b2.reshape(1, OUT), g2.reshape(1, OUT),
      be2.reshape(1, OUT), minv.reshape(BN, C2), s.reshape(BN, C2),
      q.reshape(BN, C2))
    return out.reshape(B, N, OUT)


# speedup vs baseline: 1.4191x; 1.3009x over previous
import functools

import jax
import jax.numpy as jnp
from jax.experimental import pallas as pl
from jax.experimental.pallas import tpu as pltpu

_K = 32
_T = 256
_BIG = 3e38


def _stage1_kernel(xt_ref, xf_ref, wbt_ref, minv_ref, s_ref, q_ref, usc):
    xt = xt_ref[0]
    xf = xf_ref[0]
    T = xt.shape[0]
    N = xf.shape[0]
    x2t = jnp.sum(xt * xt, axis=1, keepdims=True)
    x2f = jnp.sum(xf * xf, axis=1, keepdims=True)
    cross = jax.lax.dot_general(
        xt, xf, (((1,), (1,)), ((), ())),
        preferred_element_type=jnp.float32)
    d = x2t + jnp.transpose(x2f) - 2.0 * cross
    bits = jax.lax.bitcast_convert_type(d, jnp.int32)
    u = bits ^ ((bits >> 31) & jnp.int32(0x7FFFFFFF))
    usc[...] = u
    kf = jnp.float32(_K)
    f = u
    w = N
    while w > 2 * _K:
        w //= 2
        f = jnp.minimum(f[:, :w], f[:, w:])
    lo = jnp.min(f, axis=1, keepdims=True) - 1
    hi = jnp.max(f, axis=1, keepdims=True)

    r64 = jax.lax.broadcasted_iota(jnp.int32, (T, 2 * _K), 0) & (2 * _K - 1)
    lane = jax.lax.broadcasted_iota(jnp.int32, (T, 2 * _K), 1)
    fx = jnp.where(lane == r64, jnp.int32(0x7FFFFFFF), f)
    p = jnp.min(fx, axis=1, keepdims=True) - 1
    cp = jnp.sum((usc[...] <= p).astype(jnp.float32), axis=1, keepdims=True)
    lo = jnp.where(cp < kf, p, lo)
    hi = jnp.where(cp >= kf, p, hi)

    def avg(a, b):
        return (a >> 1) + (b >> 1) + (a & b & 1)

    def body(carry):
        i, lo, hi = carry
        m = avg(lo, hi)
        c = jnp.sum((usc[...] <= m).astype(jnp.float32), axis=1, keepdims=True)
        hi = jnp.where(c >= kf, m, hi)
        lo = jnp.where(c >= kf, lo, m)
        return i + 1, lo, hi

    carry = (jnp.int32(0), lo, hi)
    for _ in range(26):
        carry = body(carry)

    def cond(carry):
        i, lo, hi = carry
        return jnp.any((hi - lo) > 1)

    _, lo, hi = jax.lax.while_loop(cond, body, carry)
    t = hi
    M = (usc[...] <= t).astype(jnp.float32)
    vT = jax.lax.dot_general(
        wbt_ref[...], xf, (((1,), (1,)), ((), ())),
        preferred_element_type=jnp.float32)
    s_ref[0] = jax.lax.dot_general(
        M, vT, (((1,), (1,)), ((), ())),
        preferred_element_type=jnp.float32)
    q_ref[0] = jax.lax.dot_general(
        M, vT * vT, (((1,), (1,)), ((), ())),
        preferred_element_type=jnp.float32)
    keep = M > 0.0
    cols = []
    for c in range(vT.shape[0]):
        vc = jax.lax.slice_in_dim(vT, c, c + 1, axis=0)
        cols.append(jnp.min(jnp.where(keep, vc, _BIG), axis=1, keepdims=True))
    minv_ref[0] = jnp.concatenate(cols, axis=1)


def _stage2_kernel(x_ref, wab_ref, b1_ref, g1_ref, be1_ref, w2_ref, b2_ref,
                   g2_ref, be2_ref, minv_ref, s_ref, q_ref, out_ref):
    x = x_ref[...]
    u = jax.lax.dot_general(
        x, wab_ref[...], (((1,), (1,)), ((), ())),
        preferred_element_type=jnp.float32) + b1_ref[...]
    s = s_ref[...]
    q = q_ref[...]
    minv = minv_ref[...]
    cnt = jnp.float32(x.shape[0] * _K)
    kf = jnp.float32(_K)
    sum_h = kf * jnp.sum(u, axis=0, keepdims=True) - jnp.sum(
        s, axis=0, keepdims=True)
    sumsq_h = jnp.sum(kf * u * u - 2.0 * u * s + q, axis=0, keepdims=True)
    mean = sum_h / cnt
    var = sumsq_h / cnt - mean * mean
    mpool = u - minv
    hn = (mpool - mean) / jnp.sqrt(var + 1e-5)
    hn = hn * g1_ref[...] + be1_ref[...]
    e = jnp.where(hn > 0.0, hn, jnp.exp(hn) - 1.0)
    y = jax.lax.dot_general(
        e, w2_ref[...], (((1,), (1,)), ((), ())),
        preferred_element_type=jnp.float32) + b2_ref[...]
    mean2 = jnp.mean(y, axis=0, keepdims=True)
    var2 = jnp.mean((y - mean2) ** 2, axis=0, keepdims=True)
    yn = (y - mean2) / jnp.sqrt(var2 + 1e-5)
    yn = yn * g2_ref[...] + be2_ref[...]
    out_ref[...] = jnp.where(yn > 0.0, yn, jnp.exp(yn) - 1.0)


@jax.jit
def kernel(x, W1, b1, g1, be1, W2, b2, g2, be2):
    B, N, C = x.shape
    C2 = W1.shape[0]
    OUT = W2.shape[0]
    T = _T
    wb = W1[:, C:]
    wab = W1[:, :C] + wb

    grid = (B, N // T)
    minv, s, q = pl.pallas_call(
        _stage1_kernel,
        grid=grid,
        in_specs=[
            pl.BlockSpec((1, T, C), lambda b, t: (b, t, 0)),
            pl.BlockSpec((1, N, C), lambda b, t: (b, 0, 0)),
            pl.BlockSpec((C2, C), lambda b, t: (0, 0)),
        ],
        out_specs=[
            pl.BlockSpec((1, T, C2), lambda b, t: (b, t, 0)),
            pl.BlockSpec((1, T, C2), lambda b, t: (b, t, 0)),
            pl.BlockSpec((1, T, C2), lambda b, t: (b, t, 0)),
        ],
        out_shape=[
            jax.ShapeDtypeStruct((B, N, C2), jnp.float32),
            jax.ShapeDtypeStruct((B, N, C2), jnp.float32),
            jax.ShapeDtypeStruct((B, N, C2), jnp.float32),
        ],
        scratch_shapes=[
            pltpu.VMEM((T, N), jnp.int32),
        ],
    )(x, x, wb)

    BN = B * N
    out = pl.pallas_call(
        _stage2_kernel,
        out_shape=jax.ShapeDtypeStruct((BN, OUT), jnp.float32),
    )(x.reshape(BN, C), wab, b1.reshape(1, C2), g1.reshape(1, C2),
      be1.reshape(1, C2), W2, b2.reshape(1, OUT), g2.reshape(1, OUT),
      be2.reshape(1, OUT), minv.reshape(BN, C2), s.reshape(BN, C2),
      q.reshape(BN, C2))
    return out.reshape(B, N, OUT)
